# baseline (speedup 1.0000x reference)
import functools

import jax
import jax.numpy as jnp
from jax import lax
from jax.experimental import pallas as pl
from jax.experimental.pallas import tpu as pltpu
from jax.experimental.pallas import tpu_sc as plsc

NUM_KEYS = 1000000
D = 64
CHUNK = 800


@functools.cache
def _build(B):
    info = plsc.get_sparse_core_info()
    NC, NS, L = info.num_cores, info.num_subcores, info.num_lanes
    NW = NC * NS
    b_per_w = B // NW
    assert B % (NW * CHUNK) == 0
    n_chunks = b_per_w // CHUNK
    mesh = plsc.VectorSubcoreMesh(core_axis_name="c", subcore_axis_name="s")

    @functools.partial(
        pl.kernel,
        mesh=mesh,
        compiler_params=pltpu.CompilerParams(use_tc_tiling_on_sc=False),
        out_type=jax.ShapeDtypeStruct((B, D), jnp.float32),
        scratch_types=[
            pltpu.VMEM((CHUNK,), jnp.int32),
            pltpu.VMEM((CHUNK, D), jnp.float32),
            pltpu.SemaphoreType.DMA,
        ],
    )
    def k(idx_hbm, table_hbm, out_hbm, idx_v, rows_v, sem):
        wid = lax.axis_index("s") * NC + lax.axis_index("c")
        base = wid * b_per_w

        def chunk_body(g, carry):
            off = base + g * CHUNK
            pltpu.sync_copy(idx_hbm.at[pl.ds(off, CHUNK)], idx_v)

            def clip_body(i, c):
                v = idx_v[pl.ds(i * L, L)]
                idx_v[pl.ds(i * L, L)] = jnp.clip(v, 0, NUM_KEYS - 1)
                return c

            lax.fori_loop(0, CHUNK // L, clip_body, 0)

            pltpu.async_copy(table_hbm.at[idx_v], rows_v, sem).wait()

            def row_body(j, c):
                for t in range(D // L):
                    x = rows_v[j, pl.ds(t * L, L)]
                    e = jnp.exp(jnp.abs(x) * -2.0)
                    th = (1.0 - e) / (1.0 + e)
                    rows_v[j, pl.ds(t * L, L)] = jnp.where(x < 0.0, -th, th)
                return c

            lax.fori_loop(0, CHUNK, row_body, 0)

            pltpu.sync_copy(rows_v, out_hbm.at[pl.ds(off, CHUNK)])
            return carry

        lax.fori_loop(0, n_chunks, chunk_body, 0)

    return k


@jax.jit
def kernel(key_codes, weight):
    Bt, H = key_codes.shape
    idx = key_codes.reshape(Bt * H)
    out = _build(Bt * H)(idx, weight)
    return out.reshape(Bt, H, D)


# speedup vs baseline: 1.2769x; 1.2769x over previous
import functools

import jax
import jax.numpy as jnp
from jax import lax
from jax.experimental import pallas as pl
from jax.experimental.pallas import tpu as pltpu
from jax.experimental.pallas import tpu_sc as plsc

NUM_KEYS = 1000000
D = 64
CHUNK = 640


def _tanh_chunk(buf, n_rows, L):
    def row_body(j, c):
        for t in range(D // L):
            x = buf[j, pl.ds(t * L, L)]
            e = jnp.exp(x + x)
            buf[j, pl.ds(t * L, L)] = 1.0 - 2.0 / (1.0 + e)
        return c

    lax.fori_loop(0, n_rows, row_body, 0, unroll=2)


@functools.cache
def _build(B):
    info = plsc.get_sparse_core_info()
    NC, NS, L = info.num_cores, info.num_subcores, info.num_lanes
    NW = NC * NS
    b_per_w = B // NW
    assert B % (NW * CHUNK) == 0
    n_chunks = b_per_w // CHUNK
    assert n_chunks % 2 == 0 and n_chunks >= 4
    mesh = plsc.VectorSubcoreMesh(core_axis_name="c", subcore_axis_name="s")

    @functools.partial(
        pl.kernel,
        mesh=mesh,
        compiler_params=pltpu.CompilerParams(use_tc_tiling_on_sc=False),
        out_type=jax.ShapeDtypeStruct((B, D), jnp.float32),
        scratch_types=[
            pltpu.VMEM((b_per_w,), jnp.int32),
            pltpu.VMEM((CHUNK, D), jnp.float32),
            pltpu.VMEM((CHUNK, D), jnp.float32),
            pltpu.SemaphoreType.DMA,
            pltpu.SemaphoreType.DMA,
            pltpu.SemaphoreType.DMA,
            pltpu.SemaphoreType.DMA,
        ],
    )
    def k(idx_hbm, table_hbm, out_hbm, idx_all, buf0, buf1, g0, g1, s0, s1):
        wid = lax.axis_index("s") * NC + lax.axis_index("c")
        base = wid * b_per_w
        bufs = (buf0, buf1)
        gsems = (g0, g1)
        ssems = (s0, s1)

        pltpu.sync_copy(idx_hbm.at[pl.ds(base, b_per_w)], idx_all)

        def clip_body(i, c):
            v = idx_all[pl.ds(i * L, L)]
            idx_all[pl.ds(i * L, L)] = jnp.clip(v, 0, NUM_KEYS - 1)
            return c

        lax.fori_loop(0, b_per_w // L, clip_body, 0, unroll=4)

        def start_gather(g, buf, sem):
            pltpu.async_copy(table_hbm.at[idx_all.at[pl.ds(g * CHUNK, CHUNK)]], buf, sem)

        def wait_gather(g, buf, sem):
            pltpu.make_async_copy(
                table_hbm.at[idx_all.at[pl.ds(g * CHUNK, CHUNK)]], buf, sem
            ).wait()

        def start_scatter(g, buf, sem):
            pltpu.async_copy(buf, out_hbm.at[pl.ds(base + g * CHUNK, CHUNK)], sem)

        def wait_scatter(g, buf, sem):
            pltpu.make_async_copy(
                buf, out_hbm.at[pl.ds(base + g * CHUNK, CHUNK)], sem
            ).wait()

        start_gather(0, buf0, g0)
        start_gather(1, buf1, g1)
        wait_gather(0, buf0, g0)
        _tanh_chunk(buf0, CHUNK, L)
        start_scatter(0, buf0, s0)
        wait_gather(1, buf1, g1)
        _tanh_chunk(buf1, CHUNK, L)
        start_scatter(1, buf1, s1)

        def pipe_body(h, c):
            g = 2 * h
            for b in (0, 1):
                gb = g + b
                X = bufs[b]
                wait_scatter(gb - 2, X, ssems[b])
                start_gather(gb, X, gsems[b])
            for b in (0, 1):
                gb = g + b
                X = bufs[b]
                wait_gather(gb, X, gsems[b])
                _tanh_chunk(X, CHUNK, L)
                start_scatter(gb, X, ssems[b])
            return c

        lax.fori_loop(1, n_chunks // 2, pipe_body, 0)

        wait_scatter(n_chunks - 2, buf0, s0)
        wait_scatter(n_chunks - 1, buf1, s1)

    return k


@jax.jit
def kernel(key_codes, weight):
    Bt, H = key_codes.shape
    idx = key_codes.reshape(Bt * H)
    out = _build(Bt * H)(idx, weight)
    return out.reshape(Bt, H, D)
